# Initial kernel scaffold; baseline (speedup 1.0000x reference)
import jax
import jax.numpy as jnp
from jax.experimental import pallas as pl


def kernel(x1, edge_index1, batch1, x2, edge_index2, batch2, W_e1, b_e1, W_a1, b_a1, Q1, W_e2, b_e2, W_a2, b_a2, Q2, Wq, Wk, Wv, Wo):
    raise NotImplementedError("write your pallas kernel here")


# speedup vs baseline: 4.7733x; 4.7733x over previous
import functools
import math

import jax
import jax.numpy as jnp
from jax import lax
from jax.experimental import pallas as pl
from jax.experimental.pallas import tpu as pltpu
from jax.experimental.pallas import tpu_sc as plsc

N = 10000
E = 320000
D = 128
B = 8
A = 32
H = 8
DH = D // H

NC = 2
NS = 16
NW = NC * NS
EW = E // NW
K = 80
NCHUNK = EW // K
NP = 10240
RPT = NP // NS


_SC_MESH = plsc.VectorSubcoreMesh(core_axis_name="c", subcore_axis_name="s")


@functools.partial(
    pl.kernel, mesh=_SC_MESH,
    out_type=jax.ShapeDtypeStruct((NC, NP, D), jnp.float32),
    scratch_types=[
        pltpu.VMEM_SHARED((NP, D), jnp.float32),
        pltpu.VMEM((K,), jnp.int32),
        pltpu.VMEM((K,), jnp.int32),
        pltpu.VMEM((K, D), jnp.float32),
        pltpu.SemaphoreType.DMA,
    ])
def _sc_spmm(x_hbm, src_hbm, dst_hbm, zrow_hbm, out_hbm,
             acc_sh, idx_s, idx_d, rows_v, sem):
    cid = lax.axis_index("c")
    sid = lax.axis_index("s")
    row0 = sid * RPT
    pltpu.sync_copy(zrow_hbm, acc_sh.at[pl.ds(row0, RPT)])
    plsc.subcore_barrier()

    base = (cid * NS + sid) * EW

    def body(i, carry):
        off = pl.multiple_of(base + i * K, 8)
        pltpu.sync_copy(src_hbm.at[pl.ds(off, K)], idx_s)
        pltpu.sync_copy(dst_hbm.at[pl.ds(off, K)], idx_d)
        pltpu.async_copy(x_hbm.at[idx_s], rows_v, sem).wait()
        pltpu.sync_copy(rows_v, acc_sh.at[idx_d], add=True)
        return carry

    lax.fori_loop(0, NCHUNK, body, 0)
    plsc.subcore_barrier()
    pltpu.sync_copy(acc_sh.at[pl.ds(row0, RPT)],
                    out_hbm.at[cid, pl.ds(row0, RPT)])


@functools.partial(
    pl.kernel, mesh=_SC_MESH,
    out_type=jax.ShapeDtypeStruct((NC, NP, D), jnp.float32),
    scratch_types=[
        pltpu.VMEM_SHARED((NP, D), jnp.float32),
        pltpu.VMEM((K,), jnp.int32),
        pltpu.VMEM((K, D), jnp.float32),
    ])
def _sc_deg(dst_hbm, zrow_hbm, ones_hbm, out_hbm, acc_sh, idx_d, ones_v):
    cid = lax.axis_index("c")
    sid = lax.axis_index("s")
    row0 = sid * RPT
    pltpu.sync_copy(zrow_hbm, acc_sh.at[pl.ds(row0, RPT)])
    pltpu.sync_copy(ones_hbm, ones_v)
    plsc.subcore_barrier()

    base = (cid * NS + sid) * EW

    def body(i, carry):
        off = pl.multiple_of(base + i * K, 8)
        pltpu.sync_copy(dst_hbm.at[pl.ds(off, K)], idx_d)
        pltpu.sync_copy(ones_v, acc_sh.at[idx_d], add=True)
        return carry

    lax.fori_loop(0, NCHUNK, body, 0)
    plsc.subcore_barrier()
    pltpu.sync_copy(acc_sh.at[pl.ds(row0, RPT)],
                    out_hbm.at[cid, pl.ds(row0, RPT)])


def _sc_aggregate(x, src, dst):
    zrow = jnp.zeros((RPT, D), jnp.float32)
    return _sc_spmm(x, src, dst, zrow)


def _sc_degree(dst):
    zrow = jnp.zeros((RPT, D), jnp.float32)
    ones = jnp.ones((K, D), jnp.float32)
    return _sc_deg(dst, zrow, ones)


_ROWS = 1000
_GRID = N // _ROWS


def _views_body(a_ref, d_ref, we_ref, be_ref, v0_ref, v1_ref, v2_ref):
    s = a_ref[0] + a_ref[1]
    deg = jnp.maximum(d_ref[0, :, 0:1] + d_ref[1, :, 0:1], 1.0)
    agg = s / deg
    outs = (v0_ref, v1_ref, v2_ref)
    for i in range(3):
        vi = jnp.dot(agg, we_ref[i], preferred_element_type=jnp.float32)
        outs[i][...] = jax.nn.relu(vi + be_ref[i:i + 1, :])


def _tc_views(agg_parts, deg_parts, W_e, b_e):
    return pl.pallas_call(
        _views_body,
        grid=(_GRID,),
        in_specs=[
            pl.BlockSpec((NC, _ROWS, D), lambda i: (0, i, 0)),
            pl.BlockSpec((NC, _ROWS, D), lambda i: (0, i, 0)),
            pl.BlockSpec((3, D, D), lambda i: (0, 0, 0)),
            pl.BlockSpec((3, D), lambda i: (0, 0)),
        ],
        out_specs=[pl.BlockSpec((_ROWS, D), lambda i: (i, 0))] * 3,
        out_shape=[jax.ShapeDtypeStruct((N, D), jnp.float32)] * 3,
    )(agg_parts, deg_parts, W_e, b_e)


def _hs_body(m0_ref, m1_ref, m2_ref, d_ref, wa_ref, ba_ref, q_ref,
             h_ref, s_ref):
    deg = jnp.maximum(d_ref[0, :, 0:1] + d_ref[1, :, 0:1], 1.0)
    scale = 1.0 / math.sqrt(D)
    hs = []
    ss = []
    for m_ref in (m0_ref, m1_ref, m2_ref):
        mi = (m_ref[0] + m_ref[1]) / deg
        hi = jax.nn.relu(
            jnp.dot(mi, wa_ref[...], preferred_element_type=jnp.float32)
            + ba_ref[...])
        si = lax.dot_general(hi, q_ref[...], (((1,), (1,)), ((), ())),
                             preferred_element_type=jnp.float32) * scale
        hs.append(hi)
        ss.append(si)
    h_ref[...] = jnp.concatenate(hs, axis=1)
    s_ref[...] = jnp.concatenate(ss, axis=1)


def _tc_hs(m_parts, deg_parts, W_a, b_a, Q):
    return pl.pallas_call(
        _hs_body,
        grid=(_GRID,),
        in_specs=[
            pl.BlockSpec((NC, _ROWS, D), lambda i: (0, i, 0)),
            pl.BlockSpec((NC, _ROWS, D), lambda i: (0, i, 0)),
            pl.BlockSpec((NC, _ROWS, D), lambda i: (0, i, 0)),
            pl.BlockSpec((NC, _ROWS, D), lambda i: (0, i, 0)),
            pl.BlockSpec((D, D), lambda i: (0, 0)),
            pl.BlockSpec((1, D), lambda i: (0, 0)),
            pl.BlockSpec((A, D), lambda i: (0, 0)),
        ],
        out_specs=[
            pl.BlockSpec((_ROWS, 3 * D), lambda i: (i, 0)),
            pl.BlockSpec((_ROWS, 3 * A), lambda i: (i, 0)),
        ],
        out_shape=[
            jax.ShapeDtypeStruct((N, 3 * D), jnp.float32),
            jax.ShapeDtypeStruct((N, 3 * A), jnp.float32),
        ],
    )(m_parts[0], m_parts[1], m_parts[2], deg_parts, W_a, b_a, Q)


def _smax_body(s_ref, b_ref, smax_ref):
    i = pl.program_id(0)
    s = s_ref[...]
    bid = b_ref[...]
    gids = lax.broadcasted_iota(jnp.int32, (_ROWS, B), 1)
    onehot = bid == gids
    neg_inf = jnp.float32(-jnp.inf)
    rows = []
    for g in range(B):
        mg = onehot[:, g:g + 1]
        rows.append(jnp.max(jnp.where(mg, s, neg_inf), axis=0, keepdims=True))
    blockmax = jnp.concatenate(rows, axis=0)
    prev = jnp.where(i == 0, neg_inf, smax_ref[...])
    smax_ref[...] = jnp.maximum(prev, blockmax)


def _tc_smax(s, batch2d):
    return pl.pallas_call(
        _smax_body,
        grid=(_GRID,),
        in_specs=[
            pl.BlockSpec((_ROWS, 3 * A), lambda i: (i, 0)),
            pl.BlockSpec((_ROWS, 1), lambda i: (i, 0)),
        ],
        out_specs=pl.BlockSpec((B, 3 * A), lambda i: (0, 0)),
        out_shape=jax.ShapeDtypeStruct((B, 3 * A), jnp.float32),
    )(s, batch2d)


def _pool_body(h_ref, s_ref, b_ref, smax_ref, sub_ref, den_scr, num_scr):
    i = pl.program_id(0)
    smax = smax_ref[...]
    smax = jnp.where(jnp.isfinite(smax), smax, 0.0)
    h = h_ref[...]
    s = s_ref[...]
    bid = b_ref[...]
    gids = lax.broadcasted_iota(jnp.int32, (_ROWS, B), 1)
    onehot = (bid == gids).astype(jnp.float32)
    sb = jnp.dot(onehot, smax, preferred_element_type=jnp.float32)
    w = jnp.exp(s - sb)
    dprev = jnp.where(i == 0, 0.0, den_scr[...])
    den_scr[...] = dprev + lax.dot_general(
        onehot, w, (((0,), (0,)), ((), ())),
        preferred_element_type=jnp.float32)
    for g in range(B):
        wg = w * onehot[:, g:g + 1]
        c = lax.dot_general(wg, h, (((0,), (0,)), ((), ())),
                            preferred_element_type=jnp.float32)
        nprev = jnp.where(i == 0, 0.0, num_scr[g])
        num_scr[g] = nprev + c

    @pl.when(i == _GRID - 1)
    def _():
        den = den_scr[...]
        total = jnp.zeros((B, A, D), jnp.float32)
        for v in range(3):
            numv = num_scr[:, A * v:A * (v + 1), D * v:D * (v + 1)]
            denv = den[:, A * v:A * (v + 1)]
            total = total + numv / (denv[:, :, None] + 1e-9)
        sub_ref[...] = total


def _tc_pool(h, s, batch2d, smax):
    return pl.pallas_call(
        _pool_body,
        grid=(_GRID,),
        in_specs=[
            pl.BlockSpec((_ROWS, 3 * D), lambda i: (i, 0)),
            pl.BlockSpec((_ROWS, 3 * A), lambda i: (i, 0)),
            pl.BlockSpec((_ROWS, 1), lambda i: (i, 0)),
            pl.BlockSpec((B, 3 * A), lambda i: (0, 0)),
        ],
        out_specs=pl.BlockSpec((B, A, D), lambda i: (0, 0, 0)),
        out_shape=jax.ShapeDtypeStruct((B, A, D), jnp.float32),
        scratch_shapes=[
            pltpu.VMEM((B, 3 * A), jnp.float32),
            pltpu.VMEM((B, 3 * A, 3 * D), jnp.float32),
        ],
    )(h, s, batch2d, smax)


def _attend(cls_q, cls_kv, slots_flat, wq, wk, wv, wo, hsel, rsel):
    scale = 1.0 / math.sqrt(DH)
    q = jnp.dot(cls_q, wq, preferred_element_type=jnp.float32)
    kc = jnp.dot(cls_kv, wk, preferred_element_type=jnp.float32)
    vc = jnp.dot(cls_kv, wv, preferred_element_type=jnp.float32)
    ks = jnp.dot(slots_flat, wk, preferred_element_type=jnp.float32)
    vs = jnp.dot(slots_flat, wv, preferred_element_type=jnp.float32)
    qrep = jnp.dot(rsel, q, preferred_element_type=jnp.float32)
    lc = jnp.dot(q * kc, hsel, preferred_element_type=jnp.float32) * scale
    ls = jnp.dot(qrep * ks, hsel, preferred_element_type=jnp.float32) * scale
    rows = []
    for g in range(B):
        rows.append(jnp.max(ls[A * g:A * (g + 1), :], axis=0, keepdims=True))
    m_slots = jnp.concatenate(rows, axis=0)
    m = jnp.maximum(m_slots, lc)
    wc = jnp.exp(lc - m)
    mrep = jnp.dot(rsel, m, preferred_element_type=jnp.float32)
    ws = jnp.exp(ls - mrep)
    den = wc + lax.dot_general(rsel, ws, (((0,), (0,)), ((), ())),
                               preferred_element_type=jnp.float32)
    hselt_ws = jnp.dot(ws, hsel.T, preferred_element_type=jnp.float32)
    p = hselt_ws * vs
    o_slots = lax.dot_general(rsel, p, (((0,), (0,)), ((), ())),
                              preferred_element_type=jnp.float32)
    o = (jnp.dot(wc, hsel.T, preferred_element_type=jnp.float32) * vc
         + o_slots) / jnp.dot(den, hsel.T, preferred_element_type=jnp.float32)
    return jnp.dot(o, wo, preferred_element_type=jnp.float32)


def _mha_body(s1_ref, s2_ref, wq_ref, wk_ref, wv_ref, wo_ref,
              o1_ref, o2_ref):
    s1 = s1_ref[...]
    s2 = s2_ref[...]
    f1 = s1.reshape(B * A, D)
    f2 = s2.reshape(B * A, D)
    rsel = (lax.broadcasted_iota(jnp.int32, (B * A, B), 0) // A
            == lax.broadcasted_iota(jnp.int32, (B * A, B), 1)
            ).astype(jnp.float32)
    hsel = (lax.broadcasted_iota(jnp.int32, (D, H), 0) // DH
            == lax.broadcasted_iota(jnp.int32, (D, H), 1)
            ).astype(jnp.float32)
    sum1 = lax.dot_general(rsel, f1, (((0,), (0,)), ((), ())),
                           preferred_element_type=jnp.float32)
    sum2 = lax.dot_general(rsel, f2, (((0,), (0,)), ((), ())),
                           preferred_element_type=jnp.float32)
    cls1 = sum1 - sum2
    cls2 = -cls1
    wq = wq_ref[...]
    wk = wk_ref[...]
    wv = wv_ref[...]
    wo = wo_ref[...]
    o1_ref[...] = _attend(cls1, cls2, f2, wq, wk, wv, wo, hsel, rsel)
    o2_ref[...] = _attend(cls2, cls1, f1, wq, wk, wv, wo, hsel, rsel)


def _tc_mha(sub1, sub2, Wq, Wk, Wv, Wo):
    return pl.pallas_call(
        _mha_body,
        out_shape=[jax.ShapeDtypeStruct((B, D), jnp.float32)] * 2,
    )(sub1, sub2, Wq, Wk, Wv, Wo)


def _encode_align(x, edge_index, batch, W_e, b_e, W_a, b_a, Q, token):
    src = edge_index[0]
    dst = edge_index[1]
    dst, _ = lax.optimization_barrier((dst, token))
    deg_parts = _sc_degree(dst)
    x, _ = lax.optimization_barrier((x, deg_parts))
    agg_parts = _sc_aggregate(x, src, dst)
    v0, v1, v2 = _tc_views(agg_parts, deg_parts, W_e, b_e)
    m_parts = []
    prev = v0
    for v in (v0, v1, v2):
        v, _ = lax.optimization_barrier((v, prev))
        prev = _sc_aggregate(v, src, dst)
        m_parts.append(prev)
    h, s = _tc_hs(m_parts, deg_parts, W_a, b_a.reshape(1, D), Q)
    batch2d = batch.reshape(N, 1)
    smax = _tc_smax(s, batch2d)
    return _tc_pool(h, s, batch2d, smax), prev


def kernel(x1, edge_index1, batch1, x2, edge_index2, batch2,
           W_e1, b_e1, W_a1, b_a1, Q1,
           W_e2, b_e2, W_a2, b_a2, Q2,
           Wq, Wk, Wv, Wo):
    token = jnp.zeros((8, 128), jnp.float32)
    sub1, token = _encode_align(x1, edge_index1, batch1,
                                W_e1, b_e1, W_a1, b_a1, Q1, token)
    sub2, _ = _encode_align(x2, edge_index2, batch2,
                            W_e2, b_e2, W_a2, b_a2, Q2, token)
    out1, out2 = _tc_mha(sub1, sub2, Wq, Wk, Wv, Wo)
    return out1, out2


# speedup vs baseline: 8.8001x; 1.8436x over previous
import functools
import math

import jax
import jax.numpy as jnp
from jax import lax
from jax.experimental import pallas as pl
from jax.experimental.pallas import tpu as pltpu
from jax.experimental.pallas import tpu_sc as plsc

N = 10000
E = 320000
D = 128
B = 8
A = 32
H = 8
DH = D // H

NC = 2
NS = 16
NW = NC * NS
EW = E // NW
K = 80
NCHUNK = EW // K
NP = 10240
RPT = NP // NS


_SC_MESH = plsc.VectorSubcoreMesh(core_axis_name="c", subcore_axis_name="s")


@functools.partial(
    pl.kernel, mesh=_SC_MESH,
    out_type=jax.ShapeDtypeStruct((NC, NP, D), jnp.float32),
    scratch_types=[
        pltpu.VMEM_SHARED((NP, D), jnp.float32),
        pltpu.VMEM((EW,), jnp.int32),
        pltpu.VMEM((EW,), jnp.int32),
        pltpu.VMEM((K, D), jnp.float32),
        pltpu.VMEM((K, D), jnp.float32),
        pltpu.SemaphoreType.DMA,
        pltpu.SemaphoreType.DMA,
    ])
def _sc_spmm(x_hbm, src_hbm, dst_hbm, zrow_hbm, out_hbm,
             acc_sh, srcv, dstv, rows0, rows1, sem0, sem1):
    cid = lax.axis_index("c")
    sid = lax.axis_index("s")
    row0 = sid * RPT
    pltpu.sync_copy(zrow_hbm, acc_sh.at[pl.ds(row0, RPT)])
    base = pl.multiple_of((cid * NS + sid) * EW, 8)
    pltpu.sync_copy(src_hbm.at[pl.ds(base, EW)], srcv)
    pltpu.sync_copy(dst_hbm.at[pl.ds(base, EW)], dstv)
    plsc.subcore_barrier()

    pltpu.async_copy(x_hbm.at[srcv.at[pl.ds(0, K)]], rows0, sem0)

    def body(j, carry):
        c = 2 * j
        pltpu.make_async_copy(
            x_hbm.at[srcv.at[pl.ds(c * K, K)]], rows0, sem0).wait()
        pltpu.async_copy(
            x_hbm.at[srcv.at[pl.ds((c + 1) * K, K)]], rows1, sem1)
        pltpu.sync_copy(rows0, acc_sh.at[dstv.at[pl.ds(c * K, K)]],
                        add=True)
        pltpu.make_async_copy(
            x_hbm.at[srcv.at[pl.ds((c + 1) * K, K)]], rows1, sem1).wait()
        pltpu.async_copy(
            x_hbm.at[srcv.at[pl.ds((c + 2) * K, K)]], rows0, sem0)
        pltpu.sync_copy(rows1, acc_sh.at[dstv.at[pl.ds((c + 1) * K, K)]],
                        add=True)
        return carry

    lax.fori_loop(0, (NCHUNK - 1) // 2, body, 0)
    c_last = NCHUNK - 1
    pltpu.make_async_copy(
        x_hbm.at[srcv.at[pl.ds(c_last * K, K)]], rows0, sem0).wait()
    pltpu.sync_copy(rows0, acc_sh.at[dstv.at[pl.ds(c_last * K, K)]],
                    add=True)
    plsc.subcore_barrier()
    pltpu.sync_copy(acc_sh.at[pl.ds(row0, RPT)],
                    out_hbm.at[cid, pl.ds(row0, RPT)])


@functools.partial(
    pl.kernel, mesh=_SC_MESH,
    out_type=jax.ShapeDtypeStruct((NC, NP, D), jnp.float32),
    scratch_types=[
        pltpu.VMEM_SHARED((NP, D), jnp.float32),
        pltpu.VMEM((EW,), jnp.int32),
        pltpu.VMEM((K, D), jnp.float32),
    ])
def _sc_deg(dst_hbm, zrow_hbm, ones_hbm, out_hbm, acc_sh, dstv, ones_v):
    cid = lax.axis_index("c")
    sid = lax.axis_index("s")
    row0 = sid * RPT
    pltpu.sync_copy(zrow_hbm, acc_sh.at[pl.ds(row0, RPT)])
    pltpu.sync_copy(ones_hbm, ones_v)
    base = pl.multiple_of((cid * NS + sid) * EW, 8)
    pltpu.sync_copy(dst_hbm.at[pl.ds(base, EW)], dstv)
    plsc.subcore_barrier()

    def body(i, carry):
        pltpu.sync_copy(ones_v, acc_sh.at[dstv.at[pl.ds(i * K, K)]],
                        add=True)
        return carry

    lax.fori_loop(0, NCHUNK, body, 0)
    plsc.subcore_barrier()
    pltpu.sync_copy(acc_sh.at[pl.ds(row0, RPT)],
                    out_hbm.at[cid, pl.ds(row0, RPT)])


def _sc_aggregate(x, src, dst):
    zrow = jnp.zeros((RPT, D), jnp.float32)
    return _sc_spmm(x, src, dst, zrow)


def _sc_degree(dst):
    zrow = jnp.zeros((RPT, D), jnp.float32)
    ones = jnp.ones((K, D), jnp.float32)
    return _sc_deg(dst, zrow, ones)


_ROWS = 1000
_GRID = N // _ROWS


def _views_body(a_ref, d_ref, we_ref, be_ref, v0_ref, v1_ref, v2_ref):
    s = a_ref[0] + a_ref[1]
    deg = jnp.maximum(d_ref[0, :, 0:1] + d_ref[1, :, 0:1], 1.0)
    agg = s / deg
    outs = (v0_ref, v1_ref, v2_ref)
    for i in range(3):
        vi = jnp.dot(agg, we_ref[i], preferred_element_type=jnp.float32)
        outs[i][...] = jax.nn.relu(vi + be_ref[i:i + 1, :])


def _tc_views(agg_parts, deg_parts, W_e, b_e):
    return pl.pallas_call(
        _views_body,
        grid=(_GRID,),
        in_specs=[
            pl.BlockSpec((NC, _ROWS, D), lambda i: (0, i, 0)),
            pl.BlockSpec((NC, _ROWS, D), lambda i: (0, i, 0)),
            pl.BlockSpec((3, D, D), lambda i: (0, 0, 0)),
            pl.BlockSpec((3, D), lambda i: (0, 0)),
        ],
        out_specs=[pl.BlockSpec((_ROWS, D), lambda i: (i, 0))] * 3,
        out_shape=[jax.ShapeDtypeStruct((N, D), jnp.float32)] * 3,
    )(agg_parts, deg_parts, W_e, b_e)


def _hs_body(m0_ref, m1_ref, m2_ref, d_ref, wa_ref, ba_ref, q_ref,
             h_ref, s_ref):
    deg = jnp.maximum(d_ref[0, :, 0:1] + d_ref[1, :, 0:1], 1.0)
    scale = 1.0 / math.sqrt(D)
    hs = []
    ss = []
    for m_ref in (m0_ref, m1_ref, m2_ref):
        mi = (m_ref[0] + m_ref[1]) / deg
        hi = jax.nn.relu(
            jnp.dot(mi, wa_ref[...], preferred_element_type=jnp.float32)
            + ba_ref[...])
        si = lax.dot_general(hi, q_ref[...], (((1,), (1,)), ((), ())),
                             preferred_element_type=jnp.float32) * scale
        hs.append(hi)
        ss.append(si)
    h_ref[...] = jnp.concatenate(hs, axis=1)
    s_ref[...] = jnp.concatenate(ss, axis=1)


def _tc_hs(m_parts, deg_parts, W_a, b_a, Q):
    return pl.pallas_call(
        _hs_body,
        grid=(_GRID,),
        in_specs=[
            pl.BlockSpec((NC, _ROWS, D), lambda i: (0, i, 0)),
            pl.BlockSpec((NC, _ROWS, D), lambda i: (0, i, 0)),
            pl.BlockSpec((NC, _ROWS, D), lambda i: (0, i, 0)),
            pl.BlockSpec((NC, _ROWS, D), lambda i: (0, i, 0)),
            pl.BlockSpec((D, D), lambda i: (0, 0)),
            pl.BlockSpec((1, D), lambda i: (0, 0)),
            pl.BlockSpec((A, D), lambda i: (0, 0)),
        ],
        out_specs=[
            pl.BlockSpec((_ROWS, 3 * D), lambda i: (i, 0)),
            pl.BlockSpec((_ROWS, 3 * A), lambda i: (i, 0)),
        ],
        out_shape=[
            jax.ShapeDtypeStruct((N, 3 * D), jnp.float32),
            jax.ShapeDtypeStruct((N, 3 * A), jnp.float32),
        ],
    )(m_parts[0], m_parts[1], m_parts[2], deg_parts, W_a, b_a, Q)


def _smax_body(s_ref, b_ref, smax_ref):
    i = pl.program_id(0)
    s = s_ref[...]
    bid = b_ref[...]
    gids = lax.broadcasted_iota(jnp.int32, (_ROWS, B), 1)
    onehot = bid == gids
    neg_inf = jnp.float32(-jnp.inf)
    rows = []
    for g in range(B):
        mg = onehot[:, g:g + 1]
        rows.append(jnp.max(jnp.where(mg, s, neg_inf), axis=0, keepdims=True))
    blockmax = jnp.concatenate(rows, axis=0)
    prev = jnp.where(i == 0, neg_inf, smax_ref[...])
    smax_ref[...] = jnp.maximum(prev, blockmax)


def _tc_smax(s, batch2d):
    return pl.pallas_call(
        _smax_body,
        grid=(_GRID,),
        in_specs=[
            pl.BlockSpec((_ROWS, 3 * A), lambda i: (i, 0)),
            pl.BlockSpec((_ROWS, 1), lambda i: (i, 0)),
        ],
        out_specs=pl.BlockSpec((B, 3 * A), lambda i: (0, 0)),
        out_shape=jax.ShapeDtypeStruct((B, 3 * A), jnp.float32),
    )(s, batch2d)


def _pool_body(h_ref, s_ref, b_ref, smax_ref, sub_ref, den_scr, num_scr):
    i = pl.program_id(0)
    smax = smax_ref[...]
    smax = jnp.where(jnp.isfinite(smax), smax, 0.0)
    h = h_ref[...]
    s = s_ref[...]
    bid = b_ref[...]
    gids = lax.broadcasted_iota(jnp.int32, (_ROWS, B), 1)
    onehot = (bid == gids).astype(jnp.float32)
    sb = jnp.dot(onehot, smax, preferred_element_type=jnp.float32)
    w = jnp.exp(s - sb)
    dprev = jnp.where(i == 0, 0.0, den_scr[...])
    den_scr[...] = dprev + lax.dot_general(
        onehot, w, (((0,), (0,)), ((), ())),
        preferred_element_type=jnp.float32)
    for g in range(B):
        wg = w * onehot[:, g:g + 1]
        c = lax.dot_general(wg, h, (((0,), (0,)), ((), ())),
                            preferred_element_type=jnp.float32)
        nprev = jnp.where(i == 0, 0.0, num_scr[g])
        num_scr[g] = nprev + c

    @pl.when(i == _GRID - 1)
    def _():
        den = den_scr[...]
        total = jnp.zeros((B, A, D), jnp.float32)
        for v in range(3):
            numv = num_scr[:, A * v:A * (v + 1), D * v:D * (v + 1)]
            denv = den[:, A * v:A * (v + 1)]
            total = total + numv / (denv[:, :, None] + 1e-9)
        sub_ref[...] = total


def _tc_pool(h, s, batch2d, smax):
    return pl.pallas_call(
        _pool_body,
        grid=(_GRID,),
        in_specs=[
            pl.BlockSpec((_ROWS, 3 * D), lambda i: (i, 0)),
            pl.BlockSpec((_ROWS, 3 * A), lambda i: (i, 0)),
            pl.BlockSpec((_ROWS, 1), lambda i: (i, 0)),
            pl.BlockSpec((B, 3 * A), lambda i: (0, 0)),
        ],
        out_specs=pl.BlockSpec((B, A, D), lambda i: (0, 0, 0)),
        out_shape=jax.ShapeDtypeStruct((B, A, D), jnp.float32),
        scratch_shapes=[
            pltpu.VMEM((B, 3 * A), jnp.float32),
            pltpu.VMEM((B, 3 * A, 3 * D), jnp.float32),
        ],
    )(h, s, batch2d, smax)


def _attend(cls_q, cls_kv, slots_flat, wq, wk, wv, wo, hsel, rsel):
    scale = 1.0 / math.sqrt(DH)
    q = jnp.dot(cls_q, wq, preferred_element_type=jnp.float32)
    kc = jnp.dot(cls_kv, wk, preferred_element_type=jnp.float32)
    vc = jnp.dot(cls_kv, wv, preferred_element_type=jnp.float32)
    ks = jnp.dot(slots_flat, wk, preferred_element_type=jnp.float32)
    vs = jnp.dot(slots_flat, wv, preferred_element_type=jnp.float32)
    qrep = jnp.dot(rsel, q, preferred_element_type=jnp.float32)
    lc = jnp.dot(q * kc, hsel, preferred_element_type=jnp.float32) * scale
    ls = jnp.dot(qrep * ks, hsel, preferred_element_type=jnp.float32) * scale
    rows = []
    for g in range(B):
        rows.append(jnp.max(ls[A * g:A * (g + 1), :], axis=0, keepdims=True))
    m_slots = jnp.concatenate(rows, axis=0)
    m = jnp.maximum(m_slots, lc)
    wc = jnp.exp(lc - m)
    mrep = jnp.dot(rsel, m, preferred_element_type=jnp.float32)
    ws = jnp.exp(ls - mrep)
    den = wc + lax.dot_general(rsel, ws, (((0,), (0,)), ((), ())),
                               preferred_element_type=jnp.float32)
    hselt_ws = jnp.dot(ws, hsel.T, preferred_element_type=jnp.float32)
    p = hselt_ws * vs
    o_slots = lax.dot_general(rsel, p, (((0,), (0,)), ((), ())),
                              preferred_element_type=jnp.float32)
    o = (jnp.dot(wc, hsel.T, preferred_element_type=jnp.float32) * vc
         + o_slots) / jnp.dot(den, hsel.T, preferred_element_type=jnp.float32)
    return jnp.dot(o, wo, preferred_element_type=jnp.float32)


def _mha_body(s1_ref, s2_ref, wq_ref, wk_ref, wv_ref, wo_ref,
              o1_ref, o2_ref):
    s1 = s1_ref[...]
    s2 = s2_ref[...]
    f1 = s1.reshape(B * A, D)
    f2 = s2.reshape(B * A, D)
    rsel = (lax.broadcasted_iota(jnp.int32, (B * A, B), 0) // A
            == lax.broadcasted_iota(jnp.int32, (B * A, B), 1)
            ).astype(jnp.float32)
    hsel = (lax.broadcasted_iota(jnp.int32, (D, H), 0) // DH
            == lax.broadcasted_iota(jnp.int32, (D, H), 1)
            ).astype(jnp.float32)
    sum1 = lax.dot_general(rsel, f1, (((0,), (0,)), ((), ())),
                           preferred_element_type=jnp.float32)
    sum2 = lax.dot_general(rsel, f2, (((0,), (0,)), ((), ())),
                           preferred_element_type=jnp.float32)
    cls1 = sum1 - sum2
    cls2 = -cls1
    wq = wq_ref[...]
    wk = wk_ref[...]
    wv = wv_ref[...]
    wo = wo_ref[...]
    o1_ref[...] = _attend(cls1, cls2, f2, wq, wk, wv, wo, hsel, rsel)
    o2_ref[...] = _attend(cls2, cls1, f1, wq, wk, wv, wo, hsel, rsel)


def _tc_mha(sub1, sub2, Wq, Wk, Wv, Wo):
    return pl.pallas_call(
        _mha_body,
        out_shape=[jax.ShapeDtypeStruct((B, D), jnp.float32)] * 2,
    )(sub1, sub2, Wq, Wk, Wv, Wo)


def _encode_align(x, edge_index, batch, W_e, b_e, W_a, b_a, Q, token):
    src = edge_index[0]
    dst = edge_index[1]
    dst, _ = lax.optimization_barrier((dst, token))
    deg_parts = _sc_degree(dst)
    x, _ = lax.optimization_barrier((x, deg_parts))
    agg_parts = _sc_aggregate(x, src, dst)
    v0, v1, v2 = _tc_views(agg_parts, deg_parts, W_e, b_e)
    m_parts = []
    prev = v0
    for v in (v0, v1, v2):
        v, _ = lax.optimization_barrier((v, prev))
        prev = _sc_aggregate(v, src, dst)
        m_parts.append(prev)
    h, s = _tc_hs(m_parts, deg_parts, W_a, b_a.reshape(1, D), Q)
    batch2d = batch.reshape(N, 1)
    smax = _tc_smax(s, batch2d)
    return _tc_pool(h, s, batch2d, smax), prev


def kernel(x1, edge_index1, batch1, x2, edge_index2, batch2,
           W_e1, b_e1, W_a1, b_a1, Q1,
           W_e2, b_e2, W_a2, b_a2, Q2,
           Wq, Wk, Wv, Wo):
    token = jnp.zeros((8, 128), jnp.float32)
    sub1, token = _encode_align(x1, edge_index1, batch1,
                                W_e1, b_e1, W_a1, b_a1, Q1, token)
    sub2, _ = _encode_align(x2, edge_index2, batch2,
                            W_e2, b_e2, W_a2, b_a2, Q2, token)
    out1, out2 = _tc_mha(sub1, sub2, Wq, Wk, Wv, Wo)
    return out1, out2


# speedup vs baseline: 8.8095x; 1.0011x over previous
import functools
import math

import jax
import jax.numpy as jnp
from jax import lax
from jax.experimental import pallas as pl
from jax.experimental.pallas import tpu as pltpu
from jax.experimental.pallas import tpu_sc as plsc

N = 10000
E = 320000
D = 128
B = 8
A = 32
H = 8
DH = D // H

NC = 2
NS = 16
NW = NC * NS
EW = E // NW
K = 80
NCHUNK = EW // K
NP = 10240
RPT = NP // NS


_SC_MESH = plsc.VectorSubcoreMesh(core_axis_name="c", subcore_axis_name="s")


def _make_spmm(W):
    @functools.partial(
        pl.kernel, mesh=_SC_MESH,
        out_type=jax.ShapeDtypeStruct((NC, NP, W), jnp.float32),
        scratch_types=[
            pltpu.VMEM_SHARED((NP, W), jnp.float32),
            pltpu.VMEM((EW,), jnp.int32),
            pltpu.VMEM((EW,), jnp.int32),
            pltpu.VMEM((K, W), jnp.float32),
            pltpu.VMEM((K, W), jnp.float32),
            pltpu.SemaphoreType.DMA,
            pltpu.SemaphoreType.DMA,
        ])
    def spmm(x_hbm, src_hbm, dst_hbm, zrow_hbm, out_hbm,
             acc_sh, srcv, dstv, rows0, rows1, sem0, sem1):
        cid = lax.axis_index("c")
        sid = lax.axis_index("s")
        row0 = sid * RPT
        pltpu.sync_copy(zrow_hbm, acc_sh.at[pl.ds(row0, RPT)])
        base = pl.multiple_of((cid * NS + sid) * EW, 8)
        pltpu.sync_copy(src_hbm.at[pl.ds(base, EW)], srcv)
        pltpu.sync_copy(dst_hbm.at[pl.ds(base, EW)], dstv)
        plsc.subcore_barrier()

        pltpu.async_copy(x_hbm.at[srcv.at[pl.ds(0, K)]], rows0, sem0)

        def body(j, carry):
            c = 2 * j
            pltpu.make_async_copy(
                x_hbm.at[srcv.at[pl.ds(c * K, K)]], rows0, sem0).wait()
            pltpu.async_copy(
                x_hbm.at[srcv.at[pl.ds((c + 1) * K, K)]], rows1, sem1)
            pltpu.sync_copy(rows0, acc_sh.at[dstv.at[pl.ds(c * K, K)]],
                            add=True)
            pltpu.make_async_copy(
                x_hbm.at[srcv.at[pl.ds((c + 1) * K, K)]], rows1,
                sem1).wait()
            pltpu.async_copy(
                x_hbm.at[srcv.at[pl.ds((c + 2) * K, K)]], rows0, sem0)
            pltpu.sync_copy(rows1,
                            acc_sh.at[dstv.at[pl.ds((c + 1) * K, K)]],
                            add=True)
            return carry

        lax.fori_loop(0, (NCHUNK - 1) // 2, body, 0)
        c_last = NCHUNK - 1
        pltpu.make_async_copy(
            x_hbm.at[srcv.at[pl.ds(c_last * K, K)]], rows0, sem0).wait()
        pltpu.sync_copy(rows0, acc_sh.at[dstv.at[pl.ds(c_last * K, K)]],
                        add=True)
        plsc.subcore_barrier()
        pltpu.sync_copy(acc_sh.at[pl.ds(row0, RPT)],
                        out_hbm.at[cid, pl.ds(row0, RPT)])

    return spmm


_sc_spmm = _make_spmm(D)


@functools.partial(
    pl.kernel, mesh=_SC_MESH,
    out_type=jax.ShapeDtypeStruct((NC, NP, D), jnp.float32),
    scratch_types=[
        pltpu.VMEM_SHARED((NP, D), jnp.float32),
        pltpu.VMEM((EW,), jnp.int32),
        pltpu.VMEM((K, D), jnp.float32),
    ])
def _sc_deg(dst_hbm, zrow_hbm, ones_hbm, out_hbm, acc_sh, dstv, ones_v):
    cid = lax.axis_index("c")
    sid = lax.axis_index("s")
    row0 = sid * RPT
    pltpu.sync_copy(zrow_hbm, acc_sh.at[pl.ds(row0, RPT)])
    pltpu.sync_copy(ones_hbm, ones_v)
    base = pl.multiple_of((cid * NS + sid) * EW, 8)
    pltpu.sync_copy(dst_hbm.at[pl.ds(base, EW)], dstv)
    plsc.subcore_barrier()

    def body(i, carry):
        pltpu.sync_copy(ones_v, acc_sh.at[dstv.at[pl.ds(i * K, K)]],
                        add=True)
        return carry

    lax.fori_loop(0, NCHUNK, body, 0)
    plsc.subcore_barrier()
    pltpu.sync_copy(acc_sh.at[pl.ds(row0, RPT)],
                    out_hbm.at[cid, pl.ds(row0, RPT)])


def _sc_aggregate(x, src, dst):
    zrow = jnp.zeros((RPT, D), jnp.float32)
    return _sc_spmm(x, src, dst, zrow)


def _sc_degree(dst):
    zrow = jnp.zeros((RPT, D), jnp.float32)
    ones = jnp.ones((K, D), jnp.float32)
    return _sc_deg(dst, zrow, ones)


_ROWS = 1000
_GRID = N // _ROWS


def _views_body(a_ref, d_ref, we_ref, be_ref, v0_ref, v1_ref, v2_ref):
    s = a_ref[0] + a_ref[1]
    deg = jnp.maximum(d_ref[0, :, 0:1] + d_ref[1, :, 0:1], 1.0)
    agg = s / deg
    outs = (v0_ref, v1_ref, v2_ref)
    for i in range(3):
        vi = jnp.dot(agg, we_ref[i], preferred_element_type=jnp.float32)
        outs[i][...] = jax.nn.relu(vi + be_ref[i:i + 1, :])


def _tc_views(agg_parts, deg_parts, W_e, b_e):
    return pl.pallas_call(
        _views_body,
        grid=(_GRID,),
        in_specs=[
            pl.BlockSpec((NC, _ROWS, D), lambda i: (0, i, 0)),
            pl.BlockSpec((NC, _ROWS, D), lambda i: (0, i, 0)),
            pl.BlockSpec((3, D, D), lambda i: (0, 0, 0)),
            pl.BlockSpec((3, D), lambda i: (0, 0)),
        ],
        out_specs=[pl.BlockSpec((_ROWS, D), lambda i: (i, 0))] * 3,
        out_shape=[jax.ShapeDtypeStruct((N, D), jnp.float32)] * 3,
    )(agg_parts, deg_parts, W_e, b_e)


def _hs_body(m0_ref, m1_ref, m2_ref, d_ref, wa_ref, ba_ref, q_ref,
             h_ref, s_ref):
    deg = jnp.maximum(d_ref[0, :, 0:1] + d_ref[1, :, 0:1], 1.0)
    scale = 1.0 / math.sqrt(D)
    hs = []
    ss = []
    for m_ref in (m0_ref, m1_ref, m2_ref):
        mi = (m_ref[0] + m_ref[1]) / deg
        hi = jax.nn.relu(
            jnp.dot(mi, wa_ref[...], preferred_element_type=jnp.float32)
            + ba_ref[...])
        si = lax.dot_general(hi, q_ref[...], (((1,), (1,)), ((), ())),
                             preferred_element_type=jnp.float32) * scale
        hs.append(hi)
        ss.append(si)
    h_ref[...] = jnp.concatenate(hs, axis=1)
    s_ref[...] = jnp.concatenate(ss, axis=1)


def _tc_hs(m_parts, deg_parts, W_a, b_a, Q):
    return pl.pallas_call(
        _hs_body,
        grid=(_GRID,),
        in_specs=[
            pl.BlockSpec((NC, _ROWS, D), lambda i: (0, i, 0)),
            pl.BlockSpec((NC, _ROWS, D), lambda i: (0, i, 0)),
            pl.BlockSpec((NC, _ROWS, D), lambda i: (0, i, 0)),
            pl.BlockSpec((NC, _ROWS, D), lambda i: (0, i, 0)),
            pl.BlockSpec((D, D), lambda i: (0, 0)),
            pl.BlockSpec((1, D), lambda i: (0, 0)),
            pl.BlockSpec((A, D), lambda i: (0, 0)),
        ],
        out_specs=[
            pl.BlockSpec((_ROWS, 3 * D), lambda i: (i, 0)),
            pl.BlockSpec((_ROWS, 3 * A), lambda i: (i, 0)),
        ],
        out_shape=[
            jax.ShapeDtypeStruct((N, 3 * D), jnp.float32),
            jax.ShapeDtypeStruct((N, 3 * A), jnp.float32),
        ],
    )(m_parts[0], m_parts[1], m_parts[2], deg_parts, W_a, b_a, Q)


def _smax_body(s_ref, b_ref, smax_ref):
    i = pl.program_id(0)
    s = s_ref[...]
    bid = b_ref[...]
    gids = lax.broadcasted_iota(jnp.int32, (_ROWS, B), 1)
    onehot = bid == gids
    neg_inf = jnp.float32(-jnp.inf)
    rows = []
    for g in range(B):
        mg = onehot[:, g:g + 1]
        rows.append(jnp.max(jnp.where(mg, s, neg_inf), axis=0, keepdims=True))
    blockmax = jnp.concatenate(rows, axis=0)
    prev = jnp.where(i == 0, neg_inf, smax_ref[...])
    smax_ref[...] = jnp.maximum(prev, blockmax)


def _tc_smax(s, batch2d):
    return pl.pallas_call(
        _smax_body,
        grid=(_GRID,),
        in_specs=[
            pl.BlockSpec((_ROWS, 3 * A), lambda i: (i, 0)),
            pl.BlockSpec((_ROWS, 1), lambda i: (i, 0)),
        ],
        out_specs=pl.BlockSpec((B, 3 * A), lambda i: (0, 0)),
        out_shape=jax.ShapeDtypeStruct((B, 3 * A), jnp.float32),
    )(s, batch2d)


def _pool_body(h_ref, s_ref, b_ref, smax_ref, sub_ref, den_scr, num_scr):
    i = pl.program_id(0)
    smax = smax_ref[...]
    smax = jnp.where(jnp.isfinite(smax), smax, 0.0)
    h = h_ref[...]
    s = s_ref[...]
    bid = b_ref[...]
    gids = lax.broadcasted_iota(jnp.int32, (_ROWS, B), 1)
    onehot = (bid == gids).astype(jnp.float32)
    sb = jnp.dot(onehot, smax, preferred_element_type=jnp.float32)
    w = jnp.exp(s - sb)
    dprev = jnp.where(i == 0, 0.0, den_scr[...])
    den_scr[...] = dprev + lax.dot_general(
        onehot, w, (((0,), (0,)), ((), ())),
        preferred_element_type=jnp.float32)
    for g in range(B):
        wg = w * onehot[:, g:g + 1]
        c = lax.dot_general(wg, h, (((0,), (0,)), ((), ())),
                            preferred_element_type=jnp.float32)
        nprev = jnp.where(i == 0, 0.0, num_scr[g])
        num_scr[g] = nprev + c

    @pl.when(i == _GRID - 1)
    def _():
        den = den_scr[...]
        total = jnp.zeros((B, A, D), jnp.float32)
        for v in range(3):
            numv = num_scr[:, A * v:A * (v + 1), D * v:D * (v + 1)]
            denv = den[:, A * v:A * (v + 1)]
            total = total + numv / (denv[:, :, None] + 1e-9)
        sub_ref[...] = total


def _tc_pool(h, s, batch2d, smax):
    return pl.pallas_call(
        _pool_body,
        grid=(_GRID,),
        in_specs=[
            pl.BlockSpec((_ROWS, 3 * D), lambda i: (i, 0)),
            pl.BlockSpec((_ROWS, 3 * A), lambda i: (i, 0)),
            pl.BlockSpec((_ROWS, 1), lambda i: (i, 0)),
            pl.BlockSpec((B, 3 * A), lambda i: (0, 0)),
        ],
        out_specs=pl.BlockSpec((B, A, D), lambda i: (0, 0, 0)),
        out_shape=jax.ShapeDtypeStruct((B, A, D), jnp.float32),
        scratch_shapes=[
            pltpu.VMEM((B, 3 * A), jnp.float32),
            pltpu.VMEM((B, 3 * A, 3 * D), jnp.float32),
        ],
    )(h, s, batch2d, smax)


def _attend(cls_q, cls_kv, slots_flat, wq, wk, wv, wo, hsel, rsel):
    scale = 1.0 / math.sqrt(DH)
    q = jnp.dot(cls_q, wq, preferred_element_type=jnp.float32)
    kc = jnp.dot(cls_kv, wk, preferred_element_type=jnp.float32)
    vc = jnp.dot(cls_kv, wv, preferred_element_type=jnp.float32)
    ks = jnp.dot(slots_flat, wk, preferred_element_type=jnp.float32)
    vs = jnp.dot(slots_flat, wv, preferred_element_type=jnp.float32)
    qrep = jnp.dot(rsel, q, preferred_element_type=jnp.float32)
    lc = jnp.dot(q * kc, hsel, preferred_element_type=jnp.float32) * scale
    ls = jnp.dot(qrep * ks, hsel, preferred_element_type=jnp.float32) * scale
    rows = []
    for g in range(B):
        rows.append(jnp.max(ls[A * g:A * (g + 1), :], axis=0, keepdims=True))
    m_slots = jnp.concatenate(rows, axis=0)
    m = jnp.maximum(m_slots, lc)
    wc = jnp.exp(lc - m)
    mrep = jnp.dot(rsel, m, preferred_element_type=jnp.float32)
    ws = jnp.exp(ls - mrep)
    den = wc + lax.dot_general(rsel, ws, (((0,), (0,)), ((), ())),
                               preferred_element_type=jnp.float32)
    hselt_ws = jnp.dot(ws, hsel.T, preferred_element_type=jnp.float32)
    p = hselt_ws * vs
    o_slots = lax.dot_general(rsel, p, (((0,), (0,)), ((), ())),
                              preferred_element_type=jnp.float32)
    o = (jnp.dot(wc, hsel.T, preferred_element_type=jnp.float32) * vc
         + o_slots) / jnp.dot(den, hsel.T, preferred_element_type=jnp.float32)
    return jnp.dot(o, wo, preferred_element_type=jnp.float32)


def _mha_body(s1_ref, s2_ref, wq_ref, wk_ref, wv_ref, wo_ref,
              o1_ref, o2_ref):
    s1 = s1_ref[...]
    s2 = s2_ref[...]
    f1 = s1.reshape(B * A, D)
    f2 = s2.reshape(B * A, D)
    rsel = (lax.broadcasted_iota(jnp.int32, (B * A, B), 0) // A
            == lax.broadcasted_iota(jnp.int32, (B * A, B), 1)
            ).astype(jnp.float32)
    hsel = (lax.broadcasted_iota(jnp.int32, (D, H), 0) // DH
            == lax.broadcasted_iota(jnp.int32, (D, H), 1)
            ).astype(jnp.float32)
    sum1 = lax.dot_general(rsel, f1, (((0,), (0,)), ((), ())),
                           preferred_element_type=jnp.float32)
    sum2 = lax.dot_general(rsel, f2, (((0,), (0,)), ((), ())),
                           preferred_element_type=jnp.float32)
    cls1 = sum1 - sum2
    cls2 = -cls1
    wq = wq_ref[...]
    wk = wk_ref[...]
    wv = wv_ref[...]
    wo = wo_ref[...]
    o1_ref[...] = _attend(cls1, cls2, f2, wq, wk, wv, wo, hsel, rsel)
    o2_ref[...] = _attend(cls2, cls1, f1, wq, wk, wv, wo, hsel, rsel)


def _tc_mha(sub1, sub2, Wq, Wk, Wv, Wo):
    return pl.pallas_call(
        _mha_body,
        out_shape=[jax.ShapeDtypeStruct((B, D), jnp.float32)] * 2,
    )(sub1, sub2, Wq, Wk, Wv, Wo)


def _encode_align(x, edge_index, batch, W_e, b_e, W_a, b_a, Q, token):
    src = edge_index[0]
    dst = edge_index[1]
    dst, _ = lax.optimization_barrier((dst, token))
    deg_parts = _sc_degree(dst)
    x, _ = lax.optimization_barrier((x, deg_parts))
    agg_parts = _sc_aggregate(x, src, dst)
    v0, v1, v2 = _tc_views(agg_parts, deg_parts, W_e, b_e)
    m_parts = []
    prev = v0
    for v in (v0, v1, v2):
        v, _ = lax.optimization_barrier((v, prev))
        prev = _sc_aggregate(v, src, dst)
        m_parts.append(prev)
    h, s = _tc_hs(m_parts, deg_parts, W_a, b_a.reshape(1, D), Q)
    batch2d = batch.reshape(N, 1)
    smax = _tc_smax(s, batch2d)
    return _tc_pool(h, s, batch2d, smax), prev


def kernel(x1, edge_index1, batch1, x2, edge_index2, batch2,
           W_e1, b_e1, W_a1, b_a1, Q1,
           W_e2, b_e2, W_a2, b_a2, Q2,
           Wq, Wk, Wv, Wo):
    token = jnp.zeros((8, 128), jnp.float32)
    sub1, token = _encode_align(x1, edge_index1, batch1,
                                W_e1, b_e1, W_a1, b_a1, Q1, token)
    sub2, _ = _encode_align(x2, edge_index2, batch2,
                            W_e2, b_e2, W_a2, b_a2, Q2, token)
    out1, out2 = _tc_mha(sub1, sub2, Wq, Wk, Wv, Wo)
    return out1, out2
